# baseline (speedup 1.0000x reference)
import functools

import jax
import jax.numpy as jnp
from jax import lax
from jax.experimental import pallas as pl
from jax.experimental.pallas import tpu as pltpu
from jax.experimental.pallas import tpu_sc as plsc

B = 16384
EMB = 256
HID = 512
D_STUDIO = EMB
D_DIST = EMB // 2
D_PROD = EMB // 2
D_TYPE = EMB // 4
N_TYPE_PAD = 16

NC = 2
NS = 16
NW = NC * NS
CHUNK = 64
NSPLIT = 1

_DIMS = (D_STUDIO, D_DIST, D_PROD)


def _sc_gather_all(nb, studio_ids, dist_ids, prod_ids,
                   studio_t, dist_t, prod_t):
    b_per_w = nb // NW
    n_chunks = b_per_w // CHUNK
    mesh = plsc.VectorSubcoreMesh(core_axis_name="c", subcore_axis_name="s")

    @functools.partial(
        pl.kernel,
        mesh=mesh,
        out_type=[
            jax.ShapeDtypeStruct((nb, d), jnp.float32) for d in _DIMS
        ],
        scratch_types=(
            [pltpu.VMEM((b_per_w,), jnp.int32) for _ in range(3)]
            + [pltpu.VMEM((CHUNK, d), jnp.float32) for d in _DIMS
               for _ in range(2)]
            + [pltpu.SemaphoreType.DMA for _ in range(7)]
        ),
    )
    def k(sid_hbm, did_hbm, pid_hbm,
          st_hbm, dt_hbm, pt_hbm,
          out_s, out_d, out_p,
          idx_s, idx_d, idx_p,
          s0, s1, d0, d1, p0, p1,
          isem, gs, gd, gp, ws, wd, wp):
        N_CHUNKS = n_chunks
        wid = lax.axis_index("s") * NC + lax.axis_index("c")
        wbase = wid * b_per_w
        wsl = pl.ds(wbase, b_per_w)

        i1 = pltpu.async_copy(sid_hbm.at[wsl], idx_s, isem)
        i2 = pltpu.async_copy(did_hbm.at[wsl], idx_d, isem)
        i3 = pltpu.async_copy(pid_hbm.at[wsl], idx_p, isem)
        i1.wait(); i2.wait(); i3.wait()

        tabs = [
            (st_hbm, idx_s, (s0, s1), out_s, gs, ws),
            (dt_hbm, idx_d, (d0, d1), out_d, gd, wd),
            (pt_hbm, idx_p, (p0, p1), out_p, gp, wp),
        ]

        def issue_gathers(i, b):
            for tbl, idx, bufs, _, gsem, _w in tabs:
                pltpu.async_copy(tbl.at[idx.at[pl.ds(i * CHUNK, CHUNK)]],
                                 bufs[b], gsem)

        issue_gathers(0, 0)

        @pl.loop(0, N_CHUNKS, step=2)
        def _(it):
            for b in (0, 1):
                i = it + b
                osl_prev = pl.ds(wbase + (i - 1) * CHUNK, CHUNK)
                osl = pl.ds(wbase + i * CHUNK, CHUNK)

                @pl.when(i >= 1)
                def _():
                    for _t, _i, bufs, out, _g, wsem in tabs:
                        pltpu.make_async_copy(
                            bufs[1 - b], out.at[osl_prev], wsem).wait()

                for tbl, idx, bufs, _o, gsem, _w in tabs:
                    pltpu.make_async_copy(
                        tbl.at[idx.at[pl.ds(i * CHUNK, CHUNK)]],
                        bufs[b], gsem).wait()

                @pl.when(i + 1 < N_CHUNKS)
                def _():
                    issue_gathers(i + 1, 1 - b)

                for _t, _i, bufs, out, _g, wsem in tabs:
                    pltpu.async_copy(bufs[b], out.at[osl], wsem)

        last = pl.ds(wbase + (N_CHUNKS - 1) * CHUNK, CHUNK)
        for _t, _i, bufs, out, _g, wsem in tabs:
            pltpu.make_async_copy(bufs[(N_CHUNKS - 1) % 2],
                                  out.at[last], wsem).wait()

    return k(studio_ids, dist_ids, prod_ids, studio_t, dist_t, prod_t)


BM = 4096


def _mlp_body(char_ref, s_ref, d_ref, p_ref, oh_ref,
              cW1_ref, cb1_ref, cW2_ref, cb2_ref, tt_ref,
              fW1_ref, fb1_ref, fW2_ref, fb2_ref, out_ref):
    f32 = jnp.float32
    bf16 = jnp.bfloat16
    dot = lambda a, b: lax.dot_general(a, b, (((1,), (0,)), ((), ())),
                                       preferred_element_type=f32)
    gelu = lambda x: x * (0.5 * (1.0 + lax.erf(x * 0.7071067811865476)))
    h = gelu(dot(char_ref[...], cW1_ref[...].astype(bf16)) + cb1_ref[...])
    char_emb = (dot(h.astype(bf16), cW2_ref[...].astype(bf16))
                + cb2_ref[...]).astype(bf16)
    type_emb = dot(oh_ref[...], tt_ref[...].astype(bf16)).astype(bf16)
    combined = jnp.concatenate(
        [s_ref[...].astype(bf16), d_ref[...].astype(bf16),
         p_ref[...].astype(bf16), char_emb, type_emb], axis=-1)
    h2 = gelu(dot(combined, fW1_ref[...].astype(bf16)) + fb1_ref[...])
    out_ref[...] = dot(h2.astype(bf16), fW2_ref[...].astype(bf16)) + fb2_ref[...]


def _tc_mlp(nb, characteristics, s_emb, d_emb, p_emb, type_onehot,
            cW1, cb1, cW2, cb2, type_table_p,
            fW1, fb1, fW2, fb2):
    grid = (nb // BM,)
    row_spec = lambda d: pl.BlockSpec((BM, d), lambda i: (i, 0))
    full_spec = lambda a: pl.BlockSpec(a.shape, lambda i: (0, 0))
    return pl.pallas_call(
        _mlp_body,
        grid=grid,
        in_specs=[
            row_spec(16), row_spec(D_STUDIO), row_spec(D_DIST),
            row_spec(D_PROD), row_spec(N_TYPE_PAD),
            full_spec(cW1), full_spec(cb1), full_spec(cW2), full_spec(cb2),
            full_spec(type_table_p),
            full_spec(fW1), full_spec(fb1), full_spec(fW2), full_spec(fb2),
        ],
        out_specs=row_spec(HID),
        out_shape=jax.ShapeDtypeStruct((nb, HID), jnp.float32),
    )(characteristics, s_emb, d_emb, p_emb, type_onehot,
      cW1, cb1, cW2, cb2, type_table_p,
      fW1, fb1, fW2, fb2)


def kernel(studio_ids, distributor_ids, prod_company_ids, characteristics,
           studio_types, studio_table, dist_table, prod_table, type_table,
           char_W1, char_b1, char_W2, char_b2,
           fus_W1, fus_b1, fus_W2, fus_b2):
    bf16 = jnp.bfloat16
    type_table_p = jnp.pad(type_table, ((0, N_TYPE_PAD - type_table.shape[0]),
                                        (0, 0)))
    fW1 = fus_W1
    fW2 = fus_W2
    cb1 = char_b1.reshape(1, -1)
    cb2 = char_b2.reshape(1, -1)
    fb1 = fus_b1.reshape(1, -1)
    fb2 = fus_b2.reshape(1, -1)
    char_bf = characteristics.astype(bf16)
    types_oh = (studio_types[:, None]
                == jnp.arange(N_TYPE_PAD, dtype=jnp.int32)[None, :]
                ).astype(bf16)

    nb = B // NSPLIT
    gathered = []
    for i in range(NSPLIT):
        sl = slice(i * nb, (i + 1) * nb)
        gathered.append(_sc_gather_all(
            nb, studio_ids[sl], distributor_ids[sl], prod_company_ids[sl],
            studio_table, dist_table, prod_table))
    outs = []
    for i in range(NSPLIT):
        sl = slice(i * nb, (i + 1) * nb)
        s_emb, d_emb, p_emb = gathered[i]
        outs.append(_tc_mlp(
            nb, char_bf[sl], s_emb, d_emb, p_emb, types_oh[sl],
            char_W1, cb1, char_W2, cb2, type_table_p, fW1, fb1, fW2, fb2))
    return jnp.concatenate(outs, axis=0) if NSPLIT > 1 else outs[0]


# speedup vs baseline: 1.0295x; 1.0295x over previous
import functools

import jax
import jax.numpy as jnp
from jax import lax
from jax.experimental import pallas as pl
from jax.experimental.pallas import tpu as pltpu
from jax.experimental.pallas import tpu_sc as plsc

B = 16384
EMB = 256
HID = 512
D_STUDIO = EMB
D_DIST = EMB // 2
D_PROD = EMB // 2
D_TYPE = EMB // 4
N_TYPE_PAD = 16

NC = 2
NS = 16
NW = NC * NS
CHUNK = 64
NSPLIT = 1

_DIMS = (D_STUDIO, D_DIST, D_PROD)


def _sc_gather_all(nb, studio_ids, dist_ids, prod_ids,
                   studio_t, dist_t, prod_t):
    b_per_w = nb // NW
    n_chunks = b_per_w // CHUNK
    mesh = plsc.VectorSubcoreMesh(core_axis_name="c", subcore_axis_name="s")

    @functools.partial(
        pl.kernel,
        mesh=mesh,
        out_type=[
            jax.ShapeDtypeStruct((nb, d), jnp.float32) for d in _DIMS
        ],
        scratch_types=(
            [pltpu.VMEM((b_per_w,), jnp.int32) for _ in range(3)]
            + [pltpu.VMEM((CHUNK, d), jnp.float32) for d in _DIMS
               for _ in range(2)]
            + [pltpu.SemaphoreType.DMA for _ in range(7)]
        ),
    )
    def k(sid_hbm, did_hbm, pid_hbm,
          st_hbm, dt_hbm, pt_hbm,
          out_s, out_d, out_p,
          idx_s, idx_d, idx_p,
          s0, s1, d0, d1, p0, p1,
          isem, gs, gd, gp, ws, wd, wp):
        N_CHUNKS = n_chunks
        wid = lax.axis_index("s") * NC + lax.axis_index("c")
        wbase = wid * b_per_w
        wsl = pl.ds(wbase, b_per_w)

        i1 = pltpu.async_copy(sid_hbm.at[wsl], idx_s, isem)
        i2 = pltpu.async_copy(did_hbm.at[wsl], idx_d, isem)
        i3 = pltpu.async_copy(pid_hbm.at[wsl], idx_p, isem)
        i1.wait(); i2.wait(); i3.wait()

        tabs = [
            (st_hbm, idx_s, (s0, s1), out_s, gs, ws),
            (dt_hbm, idx_d, (d0, d1), out_d, gd, wd),
            (pt_hbm, idx_p, (p0, p1), out_p, gp, wp),
        ]

        def issue_gathers(i, b):
            for tbl, idx, bufs, _, gsem, _w in tabs:
                pltpu.async_copy(tbl.at[idx.at[pl.ds(i * CHUNK, CHUNK)]],
                                 bufs[b], gsem)

        issue_gathers(0, 0)

        @pl.loop(0, N_CHUNKS, step=2)
        def _(it):
            for b in (0, 1):
                i = it + b
                osl_prev = pl.ds(wbase + (i - 1) * CHUNK, CHUNK)
                osl = pl.ds(wbase + i * CHUNK, CHUNK)

                @pl.when(i >= 1)
                def _():
                    for _t, _i, bufs, out, _g, wsem in tabs:
                        pltpu.make_async_copy(
                            bufs[1 - b], out.at[osl_prev], wsem).wait()

                for tbl, idx, bufs, _o, gsem, _w in tabs:
                    pltpu.make_async_copy(
                        tbl.at[idx.at[pl.ds(i * CHUNK, CHUNK)]],
                        bufs[b], gsem).wait()

                @pl.when(i + 1 < N_CHUNKS)
                def _():
                    issue_gathers(i + 1, 1 - b)

                for _t, _i, bufs, out, _g, wsem in tabs:
                    pltpu.async_copy(bufs[b], out.at[osl], wsem)

        last = pl.ds(wbase + (N_CHUNKS - 1) * CHUNK, CHUNK)
        for _t, _i, bufs, out, _g, wsem in tabs:
            pltpu.make_async_copy(bufs[(N_CHUNKS - 1) % 2],
                                  out.at[last], wsem).wait()

    return k(studio_ids, dist_ids, prod_ids, studio_t, dist_t, prod_t)


BM = 2048


def _mlp_body(char_ref, s_ref, d_ref, p_ref, oh_ref,
              cW1_ref, cb1_ref, cW2_ref, cb2_ref, tt_ref,
              fW1_ref, fb1_ref, fW2_ref, fb2_ref, out_ref):
    f32 = jnp.float32
    bf16 = jnp.bfloat16
    dot = lambda a, b: lax.dot_general(a, b, (((1,), (0,)), ((), ())),
                                       preferred_element_type=f32)
    gelu = lambda x: x * (0.5 * (1.0 + lax.erf(x * 0.7071067811865476)))
    h = gelu(dot(char_ref[...], cW1_ref[...].astype(bf16)) + cb1_ref[...])
    char_emb = (dot(h.astype(bf16), cW2_ref[...].astype(bf16))
                + cb2_ref[...]).astype(bf16)
    type_emb = dot(oh_ref[...], tt_ref[...].astype(bf16)).astype(bf16)
    combined = jnp.concatenate(
        [s_ref[...].astype(bf16), d_ref[...].astype(bf16),
         p_ref[...].astype(bf16), char_emb, type_emb], axis=-1)
    h2 = gelu(dot(combined, fW1_ref[...].astype(bf16)) + fb1_ref[...])
    out_ref[...] = dot(h2.astype(bf16), fW2_ref[...].astype(bf16)) + fb2_ref[...]


def _tc_mlp(nb, characteristics, s_emb, d_emb, p_emb, type_onehot,
            cW1, cb1, cW2, cb2, type_table_p,
            fW1, fb1, fW2, fb2):
    grid = (nb // BM,)
    row_spec = lambda d: pl.BlockSpec((BM, d), lambda i: (i, 0))
    full_spec = lambda a: pl.BlockSpec(a.shape, lambda i: (0, 0))
    return pl.pallas_call(
        _mlp_body,
        grid=grid,
        in_specs=[
            row_spec(16), row_spec(D_STUDIO), row_spec(D_DIST),
            row_spec(D_PROD), row_spec(N_TYPE_PAD),
            full_spec(cW1), full_spec(cb1), full_spec(cW2), full_spec(cb2),
            full_spec(type_table_p),
            full_spec(fW1), full_spec(fb1), full_spec(fW2), full_spec(fb2),
        ],
        out_specs=row_spec(HID),
        out_shape=jax.ShapeDtypeStruct((nb, HID), jnp.float32),
    )(characteristics, s_emb, d_emb, p_emb, type_onehot,
      cW1, cb1, cW2, cb2, type_table_p,
      fW1, fb1, fW2, fb2)


def kernel(studio_ids, distributor_ids, prod_company_ids, characteristics,
           studio_types, studio_table, dist_table, prod_table, type_table,
           char_W1, char_b1, char_W2, char_b2,
           fus_W1, fus_b1, fus_W2, fus_b2):
    bf16 = jnp.bfloat16
    type_table_p = jnp.pad(type_table, ((0, N_TYPE_PAD - type_table.shape[0]),
                                        (0, 0)))
    fW1 = fus_W1
    fW2 = fus_W2
    cb1 = char_b1.reshape(1, -1)
    cb2 = char_b2.reshape(1, -1)
    fb1 = fus_b1.reshape(1, -1)
    fb2 = fus_b2.reshape(1, -1)
    char_bf = characteristics.astype(bf16)
    types_oh = (studio_types[:, None]
                == jnp.arange(N_TYPE_PAD, dtype=jnp.int32)[None, :]
                ).astype(bf16)

    nb = B // NSPLIT
    gathered = []
    for i in range(NSPLIT):
        sl = slice(i * nb, (i + 1) * nb)
        gathered.append(_sc_gather_all(
            nb, studio_ids[sl], distributor_ids[sl], prod_company_ids[sl],
            studio_table, dist_table, prod_table))
    outs = []
    for i in range(NSPLIT):
        sl = slice(i * nb, (i + 1) * nb)
        s_emb, d_emb, p_emb = gathered[i]
        outs.append(_tc_mlp(
            nb, char_bf[sl], s_emb, d_emb, p_emb, types_oh[sl],
            char_W1, cb1, char_W2, cb2, type_table_p, fW1, fb1, fW2, fb2))
    return jnp.concatenate(outs, axis=0) if NSPLIT > 1 else outs[0]


# speedup vs baseline: 1.0656x; 1.0350x over previous
import functools

import jax
import jax.numpy as jnp
from jax import lax
from jax.experimental import pallas as pl
from jax.experimental.pallas import tpu as pltpu
from jax.experimental.pallas import tpu_sc as plsc

B = 16384
EMB = 256
HID = 512
D_STUDIO = EMB
D_STUDIO_PK = EMB // 2
D_DIST = EMB // 2
D_PROD = EMB // 2
D_TYPE = EMB // 4
N_TYPE_PAD = 16

NC = 2
NS = 16
NW = NC * NS
CHUNK = 64
NSPLIT = 1

_DIMS = (D_STUDIO_PK, D_DIST, D_PROD)


def _sc_gather_all(nb, studio_ids, dist_ids, prod_ids,
                   studio_t, dist_t, prod_t):
    b_per_w = nb // NW
    n_chunks = b_per_w // CHUNK
    mesh = plsc.VectorSubcoreMesh(core_axis_name="c", subcore_axis_name="s")

    @functools.partial(
        pl.kernel,
        mesh=mesh,
        out_type=[
            jax.ShapeDtypeStruct((nb, d), jnp.float32) for d in _DIMS
        ],
        scratch_types=(
            [pltpu.VMEM((b_per_w,), jnp.int32) for _ in range(3)]
            + [pltpu.VMEM((CHUNK, d), jnp.float32) for d in _DIMS
               for _ in range(2)]
            + [pltpu.SemaphoreType.DMA for _ in range(7)]
        ),
    )
    def k(sid_hbm, did_hbm, pid_hbm,
          st_hbm, dt_hbm, pt_hbm,
          out_s, out_d, out_p,
          idx_s, idx_d, idx_p,
          s0, s1, d0, d1, p0, p1,
          isem, gs, gd, gp, ws, wd, wp):
        N_CHUNKS = n_chunks
        wid = lax.axis_index("s") * NC + lax.axis_index("c")
        wbase = wid * b_per_w
        wsl = pl.ds(wbase, b_per_w)

        i1 = pltpu.async_copy(sid_hbm.at[wsl], idx_s, isem)
        i2 = pltpu.async_copy(did_hbm.at[wsl], idx_d, isem)
        i3 = pltpu.async_copy(pid_hbm.at[wsl], idx_p, isem)
        i1.wait(); i2.wait(); i3.wait()

        tabs = [
            (st_hbm, idx_s, (s0, s1), out_s, gs, ws),
            (dt_hbm, idx_d, (d0, d1), out_d, gd, wd),
            (pt_hbm, idx_p, (p0, p1), out_p, gp, wp),
        ]

        def issue_gathers(i, b):
            for tbl, idx, bufs, _, gsem, _w in tabs:
                pltpu.async_copy(tbl.at[idx.at[pl.ds(i * CHUNK, CHUNK)]],
                                 bufs[b], gsem)

        issue_gathers(0, 0)

        @pl.loop(0, N_CHUNKS, step=2)
        def _(it):
            for b in (0, 1):
                i = it + b
                osl_prev = pl.ds(wbase + (i - 1) * CHUNK, CHUNK)
                osl = pl.ds(wbase + i * CHUNK, CHUNK)

                @pl.when(i >= 1)
                def _():
                    for _t, _i, bufs, out, _g, wsem in tabs:
                        pltpu.make_async_copy(
                            bufs[1 - b], out.at[osl_prev], wsem).wait()

                for tbl, idx, bufs, _o, gsem, _w in tabs:
                    pltpu.make_async_copy(
                        tbl.at[idx.at[pl.ds(i * CHUNK, CHUNK)]],
                        bufs[b], gsem).wait()

                @pl.when(i + 1 < N_CHUNKS)
                def _():
                    issue_gathers(i + 1, 1 - b)

                for _t, _i, bufs, out, _g, wsem in tabs:
                    pltpu.async_copy(bufs[b], out.at[osl], wsem)

        last = pl.ds(wbase + (N_CHUNKS - 1) * CHUNK, CHUNK)
        for _t, _i, bufs, out, _g, wsem in tabs:
            pltpu.make_async_copy(bufs[(N_CHUNKS - 1) % 2],
                                  out.at[last], wsem).wait()

    return k(studio_ids, dist_ids, prod_ids, studio_t, dist_t, prod_t)


BM = 2048


def _mlp_body(char_ref, s_ref, d_ref, p_ref, oh_ref,
              cW1_ref, cb1_ref, cW2_ref, cb2_ref, tt_ref,
              fW1_ref, fb1_ref, fW2_ref, fb2_ref, out_ref):
    f32 = jnp.float32
    bf16 = jnp.bfloat16
    dot = lambda a, b: lax.dot_general(a, b, (((1,), (0,)), ((), ())),
                                       preferred_element_type=f32)
    gelu = lambda x: x * (0.5 * (1.0 + lax.erf(x * 0.7071067811865476)))
    h = gelu(dot(char_ref[...], cW1_ref[...].astype(bf16)) + cb1_ref[...])
    char_emb = (dot(h.astype(bf16), cW2_ref[...].astype(bf16))
                + cb2_ref[...]).astype(bf16)
    type_emb = dot(oh_ref[...], tt_ref[...].astype(bf16)).astype(bf16)
    u = lax.bitcast_convert_type(s_ref[...], jnp.int32)
    s_even = lax.bitcast_convert_type(
        lax.shift_left(u, 16), f32).astype(bf16)
    s_odd = lax.bitcast_convert_type(
        jnp.bitwise_and(u, jnp.int32(-65536)), f32).astype(bf16)
    combined = jnp.concatenate(
        [s_even, s_odd, d_ref[...].astype(bf16),
         p_ref[...].astype(bf16), char_emb, type_emb], axis=-1)
    h2 = gelu(dot(combined, fW1_ref[...].astype(bf16)) + fb1_ref[...])
    out_ref[...] = dot(h2.astype(bf16), fW2_ref[...].astype(bf16)) + fb2_ref[...]


def _tc_mlp(nb, characteristics, s_emb, d_emb, p_emb, type_onehot,
            cW1, cb1, cW2, cb2, type_table_p,
            fW1, fb1, fW2, fb2):
    grid = (nb // BM,)
    row_spec = lambda d: pl.BlockSpec((BM, d), lambda i: (i, 0))
    full_spec = lambda a: pl.BlockSpec(a.shape, lambda i: (0, 0))
    return pl.pallas_call(
        _mlp_body,
        grid=grid,
        in_specs=[
            row_spec(16), row_spec(D_STUDIO_PK), row_spec(D_DIST),
            row_spec(D_PROD), row_spec(N_TYPE_PAD),
            full_spec(cW1), full_spec(cb1), full_spec(cW2), full_spec(cb2),
            full_spec(type_table_p),
            full_spec(fW1), full_spec(fb1), full_spec(fW2), full_spec(fb2),
        ],
        out_specs=row_spec(HID),
        out_shape=jax.ShapeDtypeStruct((nb, HID), jnp.float32),
    )(characteristics, s_emb, d_emb, p_emb, type_onehot,
      cW1, cb1, cW2, cb2, type_table_p,
      fW1, fb1, fW2, fb2)


def kernel(studio_ids, distributor_ids, prod_company_ids, characteristics,
           studio_types, studio_table, dist_table, prod_table, type_table,
           char_W1, char_b1, char_W2, char_b2,
           fus_W1, fus_b1, fus_W2, fus_b2):
    bf16 = jnp.bfloat16
    type_table_p = jnp.pad(type_table, ((0, N_TYPE_PAD - type_table.shape[0]),
                                        (0, 0)))
    fW1 = fus_W1
    fW2 = fus_W2
    cb1 = char_b1.reshape(1, -1)
    cb2 = char_b2.reshape(1, -1)
    fb1 = fus_b1.reshape(1, -1)
    fb2 = fus_b2.reshape(1, -1)
    char_bf = characteristics.astype(bf16)
    types_oh = (studio_types[:, None]
                == jnp.arange(N_TYPE_PAD, dtype=jnp.int32)[None, :]
                ).astype(bf16)

    nb = B // NSPLIT
    studio_pk = lax.bitcast_convert_type(
        studio_table.astype(bf16).reshape(-1, D_STUDIO_PK, 2), jnp.float32)
    fW1 = jnp.concatenate(
        [fW1[:D_STUDIO].reshape(D_STUDIO_PK, 2, HID)
         .transpose(1, 0, 2).reshape(D_STUDIO, HID),
         fW1[D_STUDIO:]], axis=0)
    gathered = []
    for i in range(NSPLIT):
        sl = slice(i * nb, (i + 1) * nb)
        gathered.append(_sc_gather_all(
            nb, studio_ids[sl], distributor_ids[sl], prod_company_ids[sl],
            studio_pk, dist_table, prod_table))
    outs = []
    for i in range(NSPLIT):
        sl = slice(i * nb, (i + 1) * nb)
        s_emb, d_emb, p_emb = gathered[i]
        outs.append(_tc_mlp(
            nb, char_bf[sl], s_emb, d_emb, p_emb, types_oh[sl],
            char_W1, cb1, char_W2, cb2, type_table_p, fW1, fb1, fW2, fb2))
    return jnp.concatenate(outs, axis=0) if NSPLIT > 1 else outs[0]
